# baseline (speedup 1.0000x reference)
import functools

import jax
import jax.numpy as jnp
from jax import lax
from jax.experimental import pallas as pl
from jax.experimental.pallas import tpu as pltpu
from jax.experimental.pallas import tpu_sc as plsc

VOCAB = 1000000
EMB = 64
SEQ = 200
BATCH = 4096
NUM_CORES = 2
NUM_SUBCORES = 16
NUM_WORKERS = NUM_CORES * NUM_SUBCORES

LANES = 128
N_UNITS = SEQ

_mesh = plsc.VectorSubcoreMesh(
    core_axis_name="c", subcore_axis_name="s",
    num_cores=NUM_CORES, num_subcores=NUM_SUBCORES)


@functools.partial(
    pl.kernel,
    out_type=jax.ShapeDtypeStruct((SEQ, 8, NUM_WORKERS, 8, LANES), jnp.float32),
    mesh=_mesh,
    scratch_types=[
        pltpu.VMEM((SEQ // 8, 8, LANES), jnp.int32),
        pltpu.VMEM((2, LANES, EMB), jnp.float32),
        pltpu.VMEM((2, 8, 8, LANES + 1), jnp.float32),
        [pltpu.SemaphoreType.DMA] * 2,
        [pltpu.SemaphoreType.DMA] * 2,
    ],
    compiler_params=pltpu.CompilerParams(use_tc_tiling_on_sc=False, needs_layout_passes=False),
)
def _gather_kernel(y4_hbm, table_hbm, out_hbm, idx_all, rows, slab, gsems, ssems):
    wid = lax.axis_index("s") * NUM_CORES + lax.axis_index("c")

    def fire_gather(u, b):
        pltpu.async_copy(
            table_hbm.at[idx_all.at[u // 8, u % 8]], rows.at[b], gsems[b])

    def drain_gather(b):
        pltpu.make_async_copy(
            table_hbm.at[pl.ds(0, LANES)], rows.at[b], gsems[b]).wait()

    def fire_store(u, b):
        pltpu.async_copy(
            slab.at[b].at[:, :, pl.ds(0, LANES)],
            out_hbm.at[u].at[:, wid], ssems[b])

    def drain_store(b):
        pltpu.make_async_copy(
            slab.at[b].at[:, :, pl.ds(0, LANES)],
            out_hbm.at[0].at[:, 0], ssems[b]).wait()

    iot = lax.iota(jnp.int32, 16)
    trv = [(cb * 16 + iot) // 8 for cb in range(EMB // 16)]
    sv = [(cb * 16 + iot) % 8 for cb in range(EMB // 16)]

    def transpose(b):
        for l0 in range(0, LANES, 8):
            lvs = [jnp.full((16,), l0 + d, jnp.int32) for d in range(8)]
            vals = [rows[b, l0 + d, pl.ds(cb * 16, 16)]
                    for d in range(8) for cb in range(EMB // 16)]
            for d in range(8):
                for cb in range(EMB // 16):
                    plsc.store_scatter(
                        slab.at[b], [trv[cb], sv[cb], lvs[d]],
                        vals[d * 4 + cb])

    pltpu.sync_copy(y4_hbm.at[:, wid], idx_all)

    fire_gather(0, 0)
    fire_gather(1, 1)

    @pl.loop(0, N_UNITS, step=2)
    def _pair(outer):
        for b in range(2):
            u = outer + b
            drain_gather(b)

            @pl.when(u >= 2)
            def _():
                drain_store(b)

            transpose(b)
            fire_store(u, b)

            @pl.when(u + 2 < N_UNITS)
            def _():
                fire_gather(u + 2, b)

    drain_store(0)
    drain_store(1)


def kernel(x, table):
    y4 = jnp.transpose(jnp.transpose(x).reshape(SEQ // 8, 8, NUM_WORKERS, LANES),
                       (0, 2, 1, 3)).astype(jnp.int32)
    o5 = _gather_kernel(y4, table)
    return o5.transpose(2, 4, 0, 1, 3).reshape(BATCH, SEQ, EMB)


# speedup vs baseline: 1.0333x; 1.0333x over previous
import functools

import jax
import jax.numpy as jnp
from jax import lax
from jax.experimental import pallas as pl
from jax.experimental.pallas import tpu as pltpu
from jax.experimental.pallas import tpu_sc as plsc

VOCAB = 1000000
EMB = 64
SEQ = 200
BATCH = 4096
NUM_CORES = 2
NUM_SUBCORES = 16
NUM_WORKERS = NUM_CORES * NUM_SUBCORES

LANES = 128
N_UNITS = SEQ

_mesh = plsc.VectorSubcoreMesh(
    core_axis_name="c", subcore_axis_name="s",
    num_cores=NUM_CORES, num_subcores=NUM_SUBCORES)


@functools.partial(
    pl.kernel,
    out_type=jax.ShapeDtypeStruct((SEQ, 8, NUM_WORKERS, 8, LANES), jnp.float32),
    mesh=_mesh,
    scratch_types=[
        pltpu.VMEM((SEQ // 8, 8, LANES), jnp.int32),
        pltpu.VMEM((2, LANES, EMB), jnp.float32),
        pltpu.VMEM((2, 8, 8, LANES + 1), jnp.float32),
        [pltpu.SemaphoreType.DMA] * 2,
        [pltpu.SemaphoreType.DMA] * 2,
    ],
    compiler_params=pltpu.CompilerParams(use_tc_tiling_on_sc=False, needs_layout_passes=False),
)
def _gather_kernel(y4_hbm, table_hbm, out_hbm, idx_all, rows, slab, gsems, ssems):
    wid = lax.axis_index("s") * NUM_CORES + lax.axis_index("c")

    def fire_gather(u, b):
        pltpu.async_copy(
            table_hbm.at[idx_all.at[u // 8, u % 8]], rows.at[b], gsems[b])

    def drain_gather(b):
        pltpu.make_async_copy(
            table_hbm.at[pl.ds(0, LANES)], rows.at[b], gsems[b]).wait()

    def fire_store(u, b):
        pltpu.async_copy(
            slab.at[b].at[:, :, pl.ds(0, LANES)],
            out_hbm.at[u].at[:, wid], ssems[b])

    def drain_store(b):
        pltpu.make_async_copy(
            slab.at[b].at[:, :, pl.ds(0, LANES)],
            out_hbm.at[0].at[:, 0], ssems[b]).wait()

    iot = lax.iota(jnp.int32, 16)
    trv = [(cb * 16 + iot) // 8 for cb in range(EMB // 16)]
    sv = [(cb * 16 + iot) % 8 for cb in range(EMB // 16)]

    def transpose(b):
        for l in range(LANES):
            lv = jnp.full((16,), l, jnp.int32)
            for cb in range(EMB // 16):
                vals = rows[b, l, pl.ds(cb * 16, 16)]
                plsc.store_scatter(slab.at[b], [trv[cb], sv[cb], lv], vals)

    pltpu.sync_copy(y4_hbm.at[:, wid], idx_all)

    fire_gather(0, 0)
    fire_gather(1, 1)

    @pl.loop(0, N_UNITS, step=2)
    def _pair(outer):
        for b in range(2):
            u = outer + b
            drain_gather(b)

            @pl.when(u >= 2)
            def _():
                drain_store(b)

            transpose(b)
            fire_store(u, b)

            @pl.when(u + 2 < N_UNITS)
            def _():
                fire_gather(u + 2, b)

    drain_store(0)
    drain_store(1)


def kernel(x, table):
    y4 = jnp.transpose(jnp.transpose(x).reshape(SEQ // 8, 8, NUM_WORKERS, LANES),
                       (0, 2, 1, 3)).astype(jnp.int32)
    o5 = _gather_kernel(y4, table)
    return o5.transpose(2, 4, 0, 1, 3).reshape(BATCH, SEQ, EMB)


# speedup vs baseline: 1.4582x; 1.4112x over previous
import functools

import jax
import jax.numpy as jnp
from jax import lax
from jax.experimental import pallas as pl
from jax.experimental.pallas import tpu as pltpu
from jax.experimental.pallas import tpu_sc as plsc

VOCAB = 1000000
EMB = 64
SEQ = 200
BATCH = 4096
NUM_CORES = 2
NUM_SUBCORES = 16
NUM_WORKERS = NUM_CORES * NUM_SUBCORES

LANES = 128
N_UNITS = SEQ

_mesh = plsc.VectorSubcoreMesh(
    core_axis_name="c", subcore_axis_name="s",
    num_cores=NUM_CORES, num_subcores=NUM_SUBCORES)


@functools.partial(
    pl.kernel,
    out_type=jax.ShapeDtypeStruct((SEQ, 8, NUM_WORKERS, 8, LANES), jnp.float32),
    mesh=_mesh,
    scratch_types=[
        pltpu.VMEM((SEQ // 8, 8, LANES), jnp.int32),
        pltpu.VMEM((2, LANES, EMB), jnp.float32),
        pltpu.VMEM((2, 8, 8, LANES + 1), jnp.float32),
        [pltpu.SemaphoreType.DMA] * 2,
        [pltpu.SemaphoreType.DMA] * 2,
    ],
    compiler_params=pltpu.CompilerParams(use_tc_tiling_on_sc=False, needs_layout_passes=False),
)
def _gather_kernel(y4_hbm, table_hbm, out_hbm, idx_all, rows, slab, gsems, ssems):
    wid = lax.axis_index("s") * NUM_CORES + lax.axis_index("c")

    def fire_gather(u, b):
        pltpu.async_copy(
            table_hbm.at[idx_all.at[u // 8, u % 8]], rows.at[b], gsems[b])

    def drain_gather(b):
        pltpu.make_async_copy(
            table_hbm.at[pl.ds(0, LANES)], rows.at[b], gsems[b]).wait()

    def fire_store(u, b):
        pltpu.async_copy(
            slab.at[b].at[:, :, pl.ds(0, LANES)],
            out_hbm.at[u].at[:, wid], ssems[b])

    def drain_store(b):
        pltpu.make_async_copy(
            slab.at[b].at[:, :, pl.ds(0, LANES)],
            out_hbm.at[0].at[:, 0], ssems[b]).wait()

    iot = lax.iota(jnp.int32, 16)
    trv = [(cb * 16 + iot) // 8 for cb in range(EMB // 16)]
    sv = [(cb * 16 + iot) % 8 for cb in range(EMB // 16)]

    def transpose(b):
        @plsc.parallel_loop(0, LANES, 1, unroll=4)
        def _l(l):
            lv = jnp.full((16,), 1, jnp.int32) * l
            for cb in range(EMB // 16):
                vals = rows[b, l, pl.ds(cb * 16, 16)]
                plsc.store_scatter(slab.at[b], [trv[cb], sv[cb], lv], vals)

    pltpu.sync_copy(y4_hbm.at[:, wid], idx_all)

    fire_gather(0, 0)
    fire_gather(1, 1)

    @pl.loop(0, N_UNITS, step=2)
    def _pair(outer):
        for b in range(2):
            u = outer + b
            drain_gather(b)

            @pl.when(u >= 2)
            def _():
                drain_store(b)

            transpose(b)
            fire_store(u, b)

            @pl.when(u + 2 < N_UNITS)
            def _():
                fire_gather(u + 2, b)

    drain_store(0)
    drain_store(1)


def kernel(x, table):
    y4 = jnp.transpose(jnp.transpose(x).reshape(SEQ // 8, 8, NUM_WORKERS, LANES),
                       (0, 2, 1, 3)).astype(jnp.int32)
    o5 = _gather_kernel(y4, table)
    return o5.transpose(2, 4, 0, 1, 3).reshape(BATCH, SEQ, EMB)
